# baseline (speedup 1.0000x reference)
import jax
import jax.numpy as jnp
import numpy as np
from jax import lax
from jax.experimental import pallas as pl
from jax.experimental.pallas import tpu as pltpu
from jax.experimental.pallas import tpu_sc as plsc

N = 16384
M = 16384
NNZ = 2684354
B = 8

NUM_CORES = 2
NUM_SUBCORES = 16
NUM_WORKERS = NUM_CORES * NUM_SUBCORES
QUADS = 2
CHUNKS = NUM_WORKERS // QUADS
VEC = 16

CH = 2048
STEPS = -(-NNZ // (CHUNKS * CH))
CPW = STEPS * CH
NNZ_PAD = CHUNKS * CPW

_HI = np.uint32(0xFFFF0000)


TPC = CH // 128


def _sc_body(xp_hbm, ind_hbm, val_hbm, part_hbm,
             xa_v, xb_v, acc0_v, acc1_v, acc2_v, acc3_v,
             sd_b0, sd_b1, val_b0, val_b1,
             sem_s0, sem_s1, sem_v0, sem_v1):
    c = lax.axis_index("c")
    s = lax.axis_index("s")
    wid = s * NUM_CORES + c
    quad = wid % QUADS
    chunk = wid // QUADS
    base = chunk * CPW

    sd_bufs = (sd_b0, sd_b1)
    val_bufs = (val_b0, val_b1)
    sems = ((sem_s0, sem_v0), (sem_s1, sem_v1))
    accs = (acc0_v, acc1_v, acc2_v, acc3_v)
    tbase = chunk * (CPW // 128)

    pltpu.sync_copy(xp_hbm.at[2 * quad], xa_v)
    pltpu.sync_copy(xp_hbm.at[2 * quad + 1], xb_v)

    zeros = jnp.zeros((VEC,), jnp.float32)

    @pl.loop(0, M // VEC, unroll=8)
    def _zero(i):
        acc0_v[pl.ds(i * VEC, VEC)] = zeros
        acc1_v[pl.ds(i * VEC, VEC)] = zeros
        acc2_v[pl.ds(i * VEC, VEC)] = zeros
        acc3_v[pl.ds(i * VEC, VEC)] = zeros

    def issue(g, bi):
        off = base + g * CH
        pltpu.make_async_copy(ind_hbm.at[pl.ds(tbase + g * TPC, TPC)],
                              sd_bufs[bi], sems[bi][0]).start()
        pltpu.make_async_copy(val_hbm.at[pl.ds(off, CH)], val_bufs[bi],
                              sems[bi][1]).start()

    def wait(bi):
        pltpu.make_async_copy(ind_hbm.at[pl.ds(0, TPC)], sd_bufs[bi],
                              sems[bi][0]).wait()
        pltpu.make_async_copy(val_hbm.at[pl.ds(0, CH)], val_bufs[bi],
                              sems[bi][1]).wait()

    def process(bi):
        sd, vb = sd_bufs[bi], val_bufs[bi]

        @pl.loop(0, TPC)
        def _tiles(t):

            @plsc.parallel_loop(0, 128, step=VEC, unroll=4)
            def _inner(j):
                si = sd[t, 0, pl.ds(j, VEC)]
                di = sd[t, 1, pl.ds(j, VEC)]
                v = vb[pl.ds(t * 128 + j, VEC)]
                for k, xt in enumerate((xa_v, xb_v)):
                    xg = plsc.load_gather(xt, [si])
                    u = plsc.bitcast(xg, jnp.uint32)
                    xhi = plsc.bitcast(u & _HI, jnp.float32)
                    xlo = plsc.bitcast(u << 16, jnp.float32)
                    plsc.addupdate_scatter(accs[2 * k], [di], v * xhi)
                    plsc.addupdate_scatter(accs[2 * k + 1], [di], v * xlo)

    issue(0, 0)

    @pl.loop(0, STEPS, step=2)
    def _outer(g):
        issue(g + 1, 1)
        wait(0)
        process(0)

        @pl.when(g + 2 < STEPS)
        def _():
            issue(g + 2, 0)

        wait(1)
        process(1)

    pltpu.sync_copy(acc0_v, part_hbm.at[chunk, 4 * quad])
    pltpu.sync_copy(acc1_v, part_hbm.at[chunk, 4 * quad + 1])
    pltpu.sync_copy(acc2_v, part_hbm.at[chunk, 4 * quad + 2])
    pltpu.sync_copy(acc3_v, part_hbm.at[chunk, 4 * quad + 3])


_sc_call = pl.kernel(
    _sc_body,
    out_type=jax.ShapeDtypeStruct((CHUNKS, B, M), jnp.float32),
    mesh=plsc.VectorSubcoreMesh(core_axis_name="c", subcore_axis_name="s",
                                num_cores=NUM_CORES,
                                num_subcores=NUM_SUBCORES),
    compiler_params=pltpu.CompilerParams(use_tc_tiling_on_sc=False,
                                         needs_layout_passes=False),
    scratch_types=[
        pltpu.VMEM((N,), jnp.float32),
        pltpu.VMEM((N,), jnp.float32),
        pltpu.VMEM((M,), jnp.float32),
        pltpu.VMEM((M,), jnp.float32),
        pltpu.VMEM((M,), jnp.float32),
        pltpu.VMEM((M,), jnp.float32),
        pltpu.VMEM((TPC, 2, 128), jnp.int32),
        pltpu.VMEM((TPC, 2, 128), jnp.int32),
        pltpu.VMEM((CH,), jnp.float32),
        pltpu.VMEM((CH,), jnp.float32),
        pltpu.SemaphoreType.DMA,
        pltpu.SemaphoreType.DMA,
        pltpu.SemaphoreType.DMA,
        pltpu.SemaphoreType.DMA,
    ],
)


def _reduce_body(p_ref, bias_ref, o_ref):
    acc = p_ref[0]
    for k in range(1, CHUNKS):
        acc = acc + p_ref[k]
    o_ref[...] = acc + bias_ref[...]


_reduce_call = pl.pallas_call(
    _reduce_body,
    out_shape=jax.ShapeDtypeStruct((B, M), jnp.float32),
)


@jax.jit
def kernel(x, values, bias, indices):
    x2 = x[..., 0]
    xu = lax.bitcast_convert_type(x2.astype(jnp.bfloat16),
                                  jnp.uint16).astype(jnp.uint32)
    xp = lax.bitcast_convert_type((xu[0::2] << 16) | xu[1::2], jnp.float32)
    pad = NNZ_PAD - NNZ
    ind_p = jnp.pad(indices, ((0, 0), (0, pad)))
    ind_t = ind_p.reshape(2, NNZ_PAD // 128, 128).transpose(1, 0, 2)
    val_p = jnp.pad(values, (0, pad))
    partials = _sc_call(xp, ind_t, val_p)
    out2 = _reduce_call(partials, bias.reshape(1, M))
    return out2[..., None]


# speedup vs baseline: 1.0675x; 1.0675x over previous
import jax
import jax.numpy as jnp
import numpy as np
from jax import lax
from jax.experimental import pallas as pl
from jax.experimental.pallas import tpu as pltpu
from jax.experimental.pallas import tpu_sc as plsc

N = 16384
M = 16384
NNZ = 2684354
B = 8

NUM_CORES = 2
NUM_SUBCORES = 16
NUM_WORKERS = NUM_CORES * NUM_SUBCORES
QUADS = 2
CHUNKS = NUM_WORKERS // QUADS
VEC = 16

CH = 2048
STEPS = -(-NNZ // (CHUNKS * CH))
CPW = STEPS * CH
NNZ_PAD = CHUNKS * CPW

_HI = np.uint32(0xFFFF0000)


TPC = CH // 128


def _sc_body(xp_hbm, ind_hbm, val_hbm, part_hbm,
             xa_v, xb_v, acc0_v, acc1_v, acc2_v, acc3_v,
             sd_b0, sd_b1, val_b0, val_b1,
             sem_s0, sem_s1, sem_v0, sem_v1):
    c = lax.axis_index("c")
    s = lax.axis_index("s")
    wid = s * NUM_CORES + c
    quad = wid % QUADS
    chunk = wid // QUADS
    base = chunk * CPW

    sd_bufs = (sd_b0, sd_b1)
    val_bufs = (val_b0, val_b1)
    sems = ((sem_s0, sem_v0), (sem_s1, sem_v1))
    accs = (acc0_v, acc1_v, acc2_v, acc3_v)
    tbase = chunk * (CPW // 128)

    pltpu.sync_copy(xp_hbm.at[2 * quad], xa_v)
    pltpu.sync_copy(xp_hbm.at[2 * quad + 1], xb_v)

    zeros = jnp.zeros((VEC,), jnp.float32)

    @pl.loop(0, M // VEC, unroll=8)
    def _zero(i):
        acc0_v[pl.ds(i * VEC, VEC)] = zeros
        acc1_v[pl.ds(i * VEC, VEC)] = zeros
        acc2_v[pl.ds(i * VEC, VEC)] = zeros
        acc3_v[pl.ds(i * VEC, VEC)] = zeros

    def issue(g, bi):
        off = base + g * CH
        pltpu.make_async_copy(ind_hbm.at[pl.ds(tbase + g * TPC, TPC)],
                              sd_bufs[bi], sems[bi][0]).start()
        pltpu.make_async_copy(val_hbm.at[pl.ds(off, CH)], val_bufs[bi],
                              sems[bi][1]).start()

    def wait(bi):
        pltpu.make_async_copy(ind_hbm.at[pl.ds(0, TPC)], sd_bufs[bi],
                              sems[bi][0]).wait()
        pltpu.make_async_copy(val_hbm.at[pl.ds(0, CH)], val_bufs[bi],
                              sems[bi][1]).wait()

    def process(bi):
        sd, vb = sd_bufs[bi], val_bufs[bi]

        @plsc.parallel_loop(0, TPC)
        def _tiles(t):

            @plsc.parallel_loop(0, 128, step=VEC, unroll=8)
            def _inner(j):
                si = sd[t, 0, pl.ds(j, VEC)]
                di = sd[t, 1, pl.ds(j, VEC)]
                v = vb[pl.ds(t * 128 + j, VEC)]
                for k, xt in enumerate((xa_v, xb_v)):
                    xg = plsc.load_gather(xt, [si])
                    u = plsc.bitcast(xg, jnp.uint32)
                    xhi = plsc.bitcast(u & _HI, jnp.float32)
                    xlo = plsc.bitcast(u << 16, jnp.float32)
                    plsc.addupdate_scatter(accs[2 * k], [di], v * xhi)
                    plsc.addupdate_scatter(accs[2 * k + 1], [di], v * xlo)

    issue(0, 0)

    @pl.loop(0, STEPS, step=2)
    def _outer(g):
        issue(g + 1, 1)
        wait(0)
        process(0)

        @pl.when(g + 2 < STEPS)
        def _():
            issue(g + 2, 0)

        wait(1)
        process(1)

    pltpu.sync_copy(acc0_v, part_hbm.at[chunk, 4 * quad])
    pltpu.sync_copy(acc1_v, part_hbm.at[chunk, 4 * quad + 1])
    pltpu.sync_copy(acc2_v, part_hbm.at[chunk, 4 * quad + 2])
    pltpu.sync_copy(acc3_v, part_hbm.at[chunk, 4 * quad + 3])


_sc_call = pl.kernel(
    _sc_body,
    out_type=jax.ShapeDtypeStruct((CHUNKS, B, M), jnp.float32),
    mesh=plsc.VectorSubcoreMesh(core_axis_name="c", subcore_axis_name="s",
                                num_cores=NUM_CORES,
                                num_subcores=NUM_SUBCORES),
    compiler_params=pltpu.CompilerParams(use_tc_tiling_on_sc=False,
                                         needs_layout_passes=False),
    scratch_types=[
        pltpu.VMEM((N,), jnp.float32),
        pltpu.VMEM((N,), jnp.float32),
        pltpu.VMEM((M,), jnp.float32),
        pltpu.VMEM((M,), jnp.float32),
        pltpu.VMEM((M,), jnp.float32),
        pltpu.VMEM((M,), jnp.float32),
        pltpu.VMEM((TPC, 2, 128), jnp.int32),
        pltpu.VMEM((TPC, 2, 128), jnp.int32),
        pltpu.VMEM((CH,), jnp.float32),
        pltpu.VMEM((CH,), jnp.float32),
        pltpu.SemaphoreType.DMA,
        pltpu.SemaphoreType.DMA,
        pltpu.SemaphoreType.DMA,
        pltpu.SemaphoreType.DMA,
    ],
)


def _reduce_body(p_ref, bias_ref, o_ref):
    acc = p_ref[0]
    for k in range(1, CHUNKS):
        acc = acc + p_ref[k]
    o_ref[...] = acc + bias_ref[...]


_reduce_call = pl.pallas_call(
    _reduce_body,
    out_shape=jax.ShapeDtypeStruct((B, M), jnp.float32),
)


@jax.jit
def kernel(x, values, bias, indices):
    x2 = x[..., 0]
    xu = lax.bitcast_convert_type(x2.astype(jnp.bfloat16),
                                  jnp.uint16).astype(jnp.uint32)
    xp = lax.bitcast_convert_type((xu[0::2] << 16) | xu[1::2], jnp.float32)
    pad = NNZ_PAD - NNZ
    ind_p = jnp.pad(indices, ((0, 0), (0, pad)))
    ind_t = ind_p.reshape(2, NNZ_PAD // 128, 128).transpose(1, 0, 2)
    val_p = jnp.pad(values, (0, pad))
    partials = _sc_call(xp, ind_t, val_p)
    out2 = _reduce_call(partials, bias.reshape(1, M))
    return out2[..., None]


# speedup vs baseline: 1.1577x; 1.0845x over previous
import jax
import jax.numpy as jnp
import numpy as np
from jax import lax
from jax.experimental import pallas as pl
from jax.experimental.pallas import tpu as pltpu
from jax.experimental.pallas import tpu_sc as plsc

N = 16384
M = 16384
NNZ = 2684354
B = 8

NUM_CORES = 2
NUM_SUBCORES = 16
NUM_WORKERS = NUM_CORES * NUM_SUBCORES
QUADS = 2
CHUNKS = NUM_WORKERS // QUADS
VEC = 16

CH = 2048
STEPS = -(-NNZ // (CHUNKS * CH))
CPW = STEPS * CH
NNZ_PAD = CHUNKS * CPW

_HI = np.uint32(0xFFFF0000)


TPC = CH // 128


def _sc_body(xp_hbm, ind_hbm, val_hbm, part_hbm,
             xa_v, xb_v, acc0_v, acc1_v, acc2_v, acc3_v,
             sd_b0, sd_b1, val_b0, val_b1,
             sem_s0, sem_s1, sem_v0, sem_v1):
    c = lax.axis_index("c")
    s = lax.axis_index("s")
    wid = s * NUM_CORES + c
    quad = wid % QUADS
    chunk = wid // QUADS
    base = chunk * CPW

    sd_bufs = (sd_b0, sd_b1)
    val_bufs = (val_b0, val_b1)
    sems = ((sem_s0, sem_v0), (sem_s1, sem_v1))
    accs = (acc0_v, acc1_v, acc2_v, acc3_v)
    tbase = chunk * (CPW // 128)

    pltpu.sync_copy(xp_hbm.at[2 * quad], xa_v)
    pltpu.sync_copy(xp_hbm.at[2 * quad + 1], xb_v)

    zeros = jnp.zeros((VEC,), jnp.float32)

    @pl.loop(0, M // VEC, unroll=8)
    def _zero(i):
        acc0_v[pl.ds(i * VEC, VEC)] = zeros
        acc1_v[pl.ds(i * VEC, VEC)] = zeros
        acc2_v[pl.ds(i * VEC, VEC)] = zeros
        acc3_v[pl.ds(i * VEC, VEC)] = zeros

    def issue(g, bi):
        off = base + g * CH
        pltpu.make_async_copy(ind_hbm.at[pl.ds(tbase + g * TPC, TPC)],
                              sd_bufs[bi], sems[bi][0]).start()
        pltpu.make_async_copy(val_hbm.at[pl.ds(off, CH)], val_bufs[bi],
                              sems[bi][1]).start()

    def wait(bi):
        pltpu.make_async_copy(ind_hbm.at[pl.ds(0, TPC)], sd_bufs[bi],
                              sems[bi][0]).wait()
        pltpu.make_async_copy(val_hbm.at[pl.ds(0, CH)], val_bufs[bi],
                              sems[bi][1]).wait()

    def process(bi):
        sd, vb = sd_bufs[bi], val_bufs[bi]

        @pl.loop(0, TPC)
        def _tiles(t):

            @plsc.parallel_loop(0, 128, step=VEC, unroll=8)
            def _inner(j):
                si = sd[t, 0, pl.ds(j, VEC)]
                di = sd[t, 1, pl.ds(j, VEC)]
                v = vb[pl.ds(t * 128 + j, VEC)]
                for k, xt in enumerate((xa_v, xb_v)):
                    xg = plsc.load_gather(xt, [si])
                    u = plsc.bitcast(xg, jnp.uint32)
                    xhi = plsc.bitcast(u & _HI, jnp.float32)
                    xlo = plsc.bitcast(u << 16, jnp.float32)
                    plsc.addupdate_scatter(accs[2 * k], [di], v * xhi)
                    plsc.addupdate_scatter(accs[2 * k + 1], [di], v * xlo)

    issue(0, 0)

    @pl.loop(0, STEPS, step=2)
    def _outer(g):
        issue(g + 1, 1)
        wait(0)
        process(0)

        @pl.when(g + 2 < STEPS)
        def _():
            issue(g + 2, 0)

        wait(1)
        process(1)

    pltpu.sync_copy(acc0_v, part_hbm.at[chunk, 4 * quad])
    pltpu.sync_copy(acc1_v, part_hbm.at[chunk, 4 * quad + 1])
    pltpu.sync_copy(acc2_v, part_hbm.at[chunk, 4 * quad + 2])
    pltpu.sync_copy(acc3_v, part_hbm.at[chunk, 4 * quad + 3])


_sc_call = pl.kernel(
    _sc_body,
    out_type=jax.ShapeDtypeStruct((CHUNKS, B, M), jnp.float32),
    mesh=plsc.VectorSubcoreMesh(core_axis_name="c", subcore_axis_name="s",
                                num_cores=NUM_CORES,
                                num_subcores=NUM_SUBCORES),
    compiler_params=pltpu.CompilerParams(use_tc_tiling_on_sc=False,
                                         needs_layout_passes=False),
    scratch_types=[
        pltpu.VMEM((N,), jnp.float32),
        pltpu.VMEM((N,), jnp.float32),
        pltpu.VMEM((M,), jnp.float32),
        pltpu.VMEM((M,), jnp.float32),
        pltpu.VMEM((M,), jnp.float32),
        pltpu.VMEM((M,), jnp.float32),
        pltpu.VMEM((TPC, 2, 128), jnp.int32),
        pltpu.VMEM((TPC, 2, 128), jnp.int32),
        pltpu.VMEM((CH,), jnp.float32),
        pltpu.VMEM((CH,), jnp.float32),
        pltpu.SemaphoreType.DMA,
        pltpu.SemaphoreType.DMA,
        pltpu.SemaphoreType.DMA,
        pltpu.SemaphoreType.DMA,
    ],
)


def _reduce_body(p_ref, bias_ref, o_ref):
    acc = p_ref[0]
    for k in range(1, CHUNKS):
        acc = acc + p_ref[k]
    o_ref[...] = acc + bias_ref[...]


_reduce_call = pl.pallas_call(
    _reduce_body,
    out_shape=jax.ShapeDtypeStruct((B, 128, 128), jnp.float32),
)


@jax.jit
def kernel(x, values, bias, indices):
    x2 = x[..., 0]
    xu = lax.bitcast_convert_type(x2.astype(jnp.bfloat16),
                                  jnp.uint16).astype(jnp.uint32)
    xp = lax.bitcast_convert_type((xu[0::2] << 16) | xu[1::2], jnp.float32)
    pad = NNZ_PAD - NNZ
    ind_p = jnp.pad(indices, ((0, 0), (0, pad)))
    ind_t = ind_p.reshape(2, NNZ_PAD // 128, 128).transpose(1, 0, 2)
    val_p = jnp.pad(values, (0, pad))
    partials = _sc_call(xp, ind_t, val_p)
    out2 = _reduce_call(partials.reshape(CHUNKS, B, 128, 128),
                        bias.reshape(1, 128, 128))
    return out2.reshape(B, M, 1)
